# baseline (speedup 1.0000x reference)
import jax
import jax.numpy as jnp
from jax import lax
from jax.experimental import pallas as pl
from jax.experimental.pallas import tpu as pltpu

EMB = 64
HID = 128
G = 8
NB = 32.0
CELL = 2.0 * NB / G
PRED = 12

_INTERPRET = False


def _pool_body(h_ref, pos_ref, posT_ref, maskH_ref, maskT_ref,
               WpT_ref, bp_ref, out_ref, gf_ref):
    f32 = jnp.float32
    n = h_ref.shape[0]
    maskH = maskH_ref[...]
    maskT = maskT_ref[...]
    rows = lax.broadcasted_iota(jnp.int32, (n, n), 0)
    cols = lax.broadcasted_iota(jnp.int32, (n, n), 1)
    neye = (rows != cols).astype(f32)
    pairm = maskH[:, 0:1] * maskT * neye

    pos = pos_ref[...]
    posT = posT_ref[...]
    rx = posT[0:1, :] - pos[:, 0:1]
    ry = posT[1:2, :] - pos[:, 1:2]
    colf = jnp.floor((rx + NB) / CELL)
    rowf = jnp.floor((ry + NB) / CELL)
    validf = (pairm
              * (jnp.abs(rx) < NB).astype(f32)
              * (jnp.abs(ry) < NB).astype(f32))
    cellf = rowf * G + colf

    h = h_ref[...]

    def cell_body(c, _):
        Mc = validf * (cellf == c.astype(f32)).astype(f32)
        gf_ref[:, pl.ds(c * HID, HID)] = jnp.dot(
            Mc, h, preferred_element_type=f32)
        return 0

    lax.fori_loop(0, G * G, cell_body, 0)
    pooled = jnp.dot(gf_ref[...], WpT_ref[...], preferred_element_type=f32)
    out_ref[...] = (pooled + bp_ref[...]) * maskH


def _social_pool_pallas(h, pos, maskH, maskT, WpT, bp2):
    n = h.shape[0]
    posT = jnp.transpose(pos)
    return pl.pallas_call(
        _pool_body,
        out_shape=jax.ShapeDtypeStruct((n, HID), jnp.float32),
        scratch_shapes=[pltpu.VMEM((n, G * G * HID), jnp.float32)],
        interpret=_INTERPRET,
    )(h, pos, posT, maskH, maskT, WpT, bp2)


def _lstm_cell(x, h, c, W_ih, W_hh, b_ih, b_hh):
    gates = x @ W_ih.T + b_ih + h @ W_hh.T + b_hh
    i, f, g, o = jnp.split(gates, 4, axis=-1)
    i = jax.nn.sigmoid(i)
    f = jax.nn.sigmoid(f)
    g = jnp.tanh(g)
    o = jax.nn.sigmoid(o)
    c2 = f * c + i * g
    h2 = o * jnp.tanh(c2)
    return h2, c2


def kernel(obs, mask, W_embed, b_embed, Wp, bp, W_ih, W_hh, b_ih, b_hh,
           W_out, b_out):
    t_obs, n, _ = obs.shape
    f32 = jnp.float32
    h = jnp.zeros((n, HID), f32)
    c = jnp.zeros((n, HID), f32)
    obs_clean = jnp.where(jnp.isnan(obs), 0.0, obs)
    mf = mask.astype(f32)[:, None]
    maskH = jnp.broadcast_to(mask.astype(f32).reshape(n, 1), (n, HID))
    maskT = mask.astype(f32).reshape(1, n)
    WpT = jnp.transpose(Wp)
    bp2 = bp.reshape(1, HID)
    for t in range(t_obs):
        pos = obs_clean[t]
        emb = jax.nn.relu(pos @ W_embed.T + b_embed)
        soc = _social_pool_pallas(h, pos, maskH, maskT, WpT, bp2)
        inp = jnp.concatenate([emb, soc], axis=-1)
        h, c = _lstm_cell(inp, h, c, W_ih, W_hh, b_ih, b_hh)
    cur_pos = obs_clean[-1] * mf
    mus, sigmas, rhos = [], [], []
    for _ in range(PRED):
        emb = jax.nn.relu(cur_pos @ W_embed.T + b_embed)
        soc = _social_pool_pallas(h, cur_pos, maskH, maskT, WpT, bp2)
        inp = jnp.concatenate([emb, soc], axis=-1)
        h, c = _lstm_cell(inp, h, c, W_ih, W_hh, b_ih, b_hh)
        raw = h @ W_out.T + b_out
        mu = raw[:, :2]
        sigma = jnp.exp(raw[:, 2:4]) + 1e-06
        rho = jnp.tanh(raw[:, 4])
        cur_pos = cur_pos + mu * mf
        mus.append(cur_pos)
        sigmas.append(sigma)
        rhos.append(rho)
    return jnp.stack(mus, 0), jnp.stack(sigmas, 0), jnp.stack(rhos, 0)
